# baseline (speedup 1.0000x reference)
import jax
import jax.numpy as jnp
from jax import lax
from jax.experimental import pallas as pl
from jax.experimental.pallas import tpu as pltpu


def _i32(x):
    return jnp.asarray(x, jnp.int32)


def _wisard_body(samples_ref, perm_ref, trained_ref, out_ref, w_ref, addr_ref):
    perm = perm_ref[0]

    PC = 256
    for pc in range(0, 2048, PC):
        iota = lax.broadcasted_iota(jnp.int32, (128, PC), 1) + _i32(pc)
        acc = jnp.zeros((128, PC), jnp.float32)
        for t in range(16):
            col = perm[:, t:t + 1]
            acc = acc + jnp.where(iota == col, jnp.float32(1 << (15 - t)),
                                  jnp.float32(0.0))
        w_ref[:, pc:pc + PC] = acc

    addr_ref[...] = lax.dot_general(
        w_ref[...], samples_ref[...],
        dimension_numbers=(((1,), (1,)), ((), ())),
        preferred_element_type=jnp.float32,
    ).astype(jnp.int32)

    RC = 8
    cnt = jnp.zeros((RC, 1024), jnp.int32)
    for rc in range(0, 128, RC):
        a = addr_ref[rc:rc + RC, :]
        m = jnp.zeros((RC, 1024), jnp.bool_)
        for k in range(64):
            tk = trained_ref[0, rc:rc + RC, k:k + 1]
            m = m | (a == tk)
        cnt = cnt + m.astype(jnp.int32)

    out_ref[0] = jnp.sum(cnt, axis=0, keepdims=True,
                         dtype=jnp.int32)


def _wisard(samples_f32, perm_i32, trained_i32, interpret=False):
    n_classes = perm_i32.shape[0]
    return pl.pallas_call(
        _wisard_body,
        grid=(n_classes,),
        in_specs=[
            pl.BlockSpec((1024, 2048), lambda i: (_i32(0), _i32(0))),
            pl.BlockSpec((1, 128, 16), lambda i: (i, _i32(0), _i32(0))),
            pl.BlockSpec((1, 128, 64), lambda i: (i, _i32(0), _i32(0))),
        ],
        out_specs=pl.BlockSpec((1, 1, 1024), lambda i: (i, _i32(0), _i32(0))),
        out_shape=jax.ShapeDtypeStruct((n_classes, 1, 1024), jnp.int32),
        scratch_shapes=[
            pltpu.VMEM((128, 2048), jnp.float32),
            pltpu.VMEM((128, 1024), jnp.int32),
        ],
        interpret=interpret,
    )(samples_f32, perm_i32, trained_i32)


def kernel(samples, tuple_mapping, trained_tuples):
    B, entry_size = samples.shape
    n_classes, n_neurons, K = trained_tuples.shape
    samples_f32 = samples.astype(jnp.int32).astype(jnp.float32)
    perm_i32 = tuple_mapping.astype(jnp.int32).reshape(n_classes, n_neurons,
                                                      entry_size // n_neurons)
    trained_i32 = trained_tuples.astype(jnp.int32)
    resp = _wisard(samples_f32, perm_i32, trained_i32)
    return resp.reshape(n_classes, B).T.astype(jnp.int8)
